# Initial kernel scaffold; baseline (speedup 1.0000x reference)
import jax
import jax.numpy as jnp
from jax.experimental import pallas as pl


def kernel(x, adj, W_pre, b_pre, Wc1, bc1, Wc2, bc2, Wc3, bc3, Wp1, bp1, Wp2, bp2):
    raise NotImplementedError("write your pallas kernel here")


# speedup vs baseline: 7.5273x; 7.5273x over previous
import functools

import jax
import jax.numpy as jnp
from jax import lax
from jax.experimental import pallas as pl
from jax.experimental.pallas import tpu as pltpu
from jax.experimental.pallas import tpu_sc as plsc

N = 10000
E = 320000
D = 128
NC = 2
NS = 16
K = 80

E_PER_T = E // (NC * NS)

ROW_START = [s * 624 for s in range(16)]
ROW_CNT = [624] * 15 + [N - 15 * 624]


@functools.lru_cache(maxsize=None)
def _mesh():
    return plsc.VectorSubcoreMesh(
        core_axis_name="c", subcore_axis_name="s", num_cores=NC, num_subcores=NS)


@functools.lru_cache(maxsize=None)
def _deg_kernel():
    @functools.partial(
        pl.kernel,
        mesh=_mesh(),
        out_type=(
            jax.ShapeDtypeStruct((N, D), jnp.float32),
            jax.ShapeDtypeStruct((N, D), jnp.float32),
            jax.ShapeDtypeStruct((N, D), jnp.float32),
            jax.ShapeDtypeStruct((N, D), jnp.float32),
        ),
        scratch_types=[
            pltpu.VMEM((K,), jnp.int32),
            pltpu.VMEM((K, D), jnp.float32),
            pltpu.VMEM_SHARED((N, D), jnp.float32),
        ],
    )
    def deg(src_hbm, dst_hbm, zeros_hbm, do_a, do_b, di_a, di_b,
            idx_v, ones_v, acc_sh):
        c = lax.axis_index("c")
        s = lax.axis_index("s")

        one16 = jnp.ones((16,), jnp.float32)

        def fill(i, _):
            for j in range(D // 16):
                ones_v[i, pl.ds(j * 16, 16)] = one16
            return 0

        lax.fori_loop(0, K, fill, 0)

        tile = c * NS + s
        base = tile * E_PER_T

        def phase(idx_hbm, out_a, out_b):
            for sid in range(NS):
                @pl.when(s == sid)
                def _():
                    r0, rn = ROW_START[sid], ROW_CNT[sid]
                    pltpu.sync_copy(zeros_hbm.at[pl.ds(r0, rn)],
                                    acc_sh.at[pl.ds(r0, rn)])

            plsc.subcore_barrier()

            def body(i, _):
                off = base + i * K
                pltpu.sync_copy(idx_hbm.at[pl.ds(off, K)], idx_v)
                pltpu.sync_copy(ones_v, acc_sh.at[idx_v], add=True)
                return 0

            lax.fori_loop(0, E_PER_T // K, body, 0)

            plsc.subcore_barrier()

            for sid in range(NS):
                @pl.when(s == sid)
                def _():
                    r0, rn = ROW_START[sid], ROW_CNT[sid]

                    @pl.when(c == 0)
                    def _():
                        pltpu.sync_copy(acc_sh.at[pl.ds(r0, rn)],
                                        out_a.at[pl.ds(r0, rn)])

                    @pl.when(c == 1)
                    def _():
                        pltpu.sync_copy(acc_sh.at[pl.ds(r0, rn)],
                                        out_b.at[pl.ds(r0, rn)])

        phase(src_hbm, do_a, do_b)
        phase(dst_hbm, di_a, di_b)

    return deg


@functools.lru_cache(maxsize=None)
def _spmm_kernel():
    @functools.partial(
        pl.kernel,
        mesh=_mesh(),
        out_type=(
            jax.ShapeDtypeStruct((N, D), jnp.float32),
            jax.ShapeDtypeStruct((N, D), jnp.float32),
        ),
        scratch_types=[
            pltpu.VMEM((K,), jnp.int32),
            pltpu.VMEM((K,), jnp.int32),
            pltpu.VMEM((K, D), jnp.float32),
            pltpu.VMEM_SHARED((N, D), jnp.float32),
            pltpu.SemaphoreType.DMA,
        ],
    )
    def spmm(src_hbm, dst_hbm, g_hbm, zeros_hbm, s_a, s_b,
             src_v, dst_v, rows_v, s_sh, sem):
        c = lax.axis_index("c")
        s = lax.axis_index("s")

        for sid in range(NS):
            @pl.when(s == sid)
            def _():
                r0, rn = ROW_START[sid], ROW_CNT[sid]
                pltpu.sync_copy(zeros_hbm.at[pl.ds(r0, rn)], s_sh.at[pl.ds(r0, rn)])

        plsc.subcore_barrier()

        tile = c * NS + s
        base = tile * E_PER_T

        def body(i, _):
            off = base + i * K
            pltpu.sync_copy(src_hbm.at[pl.ds(off, K)], src_v)
            pltpu.sync_copy(dst_hbm.at[pl.ds(off, K)], dst_v)
            pltpu.async_copy(g_hbm.at[src_v], rows_v, sem).wait()
            pltpu.sync_copy(rows_v, s_sh.at[dst_v], add=True)
            return 0

        lax.fori_loop(0, E_PER_T // K, body, 0)

        plsc.subcore_barrier()

        for sid in range(NS):
            @pl.when(s == sid)
            def _():
                r0, rn = ROW_START[sid], ROW_CNT[sid]

                @pl.when(c == 0)
                def _():
                    pltpu.sync_copy(s_sh.at[pl.ds(r0, rn)], s_a.at[pl.ds(r0, rn)])

                @pl.when(c == 1)
                def _():
                    pltpu.sync_copy(s_sh.at[pl.ds(r0, rn)], s_b.at[pl.ds(r0, rn)])

    return spmm


BLK = 1000
GRID = N // BLK


def _row_spec(w):
    return pl.BlockSpec((BLK, w), lambda i: (i, 0))


def _full_spec(shape):
    return pl.BlockSpec(shape, lambda i: tuple(0 for _ in shape))


def _pre_body(x, wpre, bpre, wc1, do_a, do_b, h_out, g_out):
    h = jnp.dot(x[...], wpre[...], preferred_element_type=jnp.float32) + bpre[...]
    h_out[...] = h
    ra = lax.rsqrt(jnp.maximum(do_a[...] + do_b[...], 1.0))[:, 0:1]
    g_out[...] = ra * jnp.dot(h, wc1[...], preferred_element_type=jnp.float32)


def _mid_body(h_in, s_a, s_b, bc, wnext, do_a, do_b, di_a, di_b, h_out, g_out):
    rb = lax.rsqrt(jnp.maximum(di_a[...] + di_b[...], 1.0))[:, 0:1]
    h = h_in[...] + jnp.maximum(rb * (s_a[...] + s_b[...]) + bc[...], 0.0)
    h_out[...] = h
    ra = lax.rsqrt(jnp.maximum(do_a[...] + do_b[...], 1.0))[:, 0:1]
    g_out[...] = ra * jnp.dot(h, wnext[...], preferred_element_type=jnp.float32)


def _post_body(h_in, s_a, s_b, bc, di_a, di_b, wp1, bp1, wp2, bp2, out):
    rb = lax.rsqrt(jnp.maximum(di_a[...] + di_b[...], 1.0))[:, 0:1]
    h = h_in[...] + jnp.maximum(rb * (s_a[...] + s_b[...]) + bc[...], 0.0)
    h2 = jnp.maximum(
        jnp.dot(h, wp1[...], preferred_element_type=jnp.float32) + bp1[...], 0.0)
    out[...] = jnp.dot(h2, wp2[...], preferred_element_type=jnp.float32) + bp2[...]


def _pre_call(x, wpre, bpre, wc1, do_a, do_b):
    return pl.pallas_call(
        _pre_body,
        grid=(GRID,),
        in_specs=[
            _row_spec(D), _full_spec((D, D)), _full_spec((D,)), _full_spec((D, D)),
            _row_spec(D), _row_spec(D),
        ],
        out_specs=[_row_spec(D), _row_spec(D)],
        out_shape=[
            jax.ShapeDtypeStruct((N, D), jnp.float32),
            jax.ShapeDtypeStruct((N, D), jnp.float32),
        ],
    )(x, wpre, bpre, wc1, do_a, do_b)


def _mid_call(h, s_a, s_b, bc, wnext, do_a, do_b, di_a, di_b):
    return pl.pallas_call(
        _mid_body,
        grid=(GRID,),
        in_specs=[
            _row_spec(D), _row_spec(D), _row_spec(D),
            _full_spec((D,)), _full_spec((D, D)),
            _row_spec(D), _row_spec(D), _row_spec(D), _row_spec(D),
        ],
        out_specs=[_row_spec(D), _row_spec(D)],
        out_shape=[
            jax.ShapeDtypeStruct((N, D), jnp.float32),
            jax.ShapeDtypeStruct((N, D), jnp.float32),
        ],
    )(h, s_a, s_b, bc, wnext, do_a, do_b, di_a, di_b)


def _post_call(h, s_a, s_b, bc, di_a, di_b, wp1, bp1, wp2, bp2):
    return pl.pallas_call(
        _post_body,
        grid=(GRID,),
        in_specs=[
            _row_spec(D), _row_spec(D), _row_spec(D),
            _full_spec((D,)),
            _row_spec(D), _row_spec(D),
            _full_spec((D, D)), _full_spec((D,)),
            _full_spec((D, D)), _full_spec((D,)),
        ],
        out_specs=_row_spec(D),
        out_shape=jax.ShapeDtypeStruct((N, D), jnp.float32),
    )(h, s_a, s_b, bc, di_a, di_b, wp1, bp1, wp2, bp2)


def kernel(x, adj, W_pre, b_pre, Wc1, bc1, Wc2, bc2, Wc3, bc3, Wp1, bp1, Wp2, bp2):
    src = adj[0]
    dst = adj[1]
    zeros128 = jnp.zeros((N, D), jnp.float32)

    do_a, do_b, di_a, di_b = _deg_kernel()(src, dst, zeros128)

    h, g = _pre_call(x, W_pre, b_pre, Wc1, do_a, do_b)

    s_a, s_b = _spmm_kernel()(src, dst, g, zeros128)
    h, g = _mid_call(h, s_a, s_b, bc1, Wc2, do_a, do_b, di_a, di_b)

    s_a, s_b = _spmm_kernel()(src, dst, g, zeros128)
    h, g = _mid_call(h, s_a, s_b, bc2, Wc3, do_a, do_b, di_a, di_b)

    s_a, s_b = _spmm_kernel()(src, dst, g, zeros128)
    out = _post_call(h, s_a, s_b, bc3, di_a, di_b, Wp1, bp1, Wp2, bp2)
    return out


# speedup vs baseline: 11.8197x; 1.5702x over previous
import functools

import jax
import jax.numpy as jnp
from jax import lax
from jax.experimental import pallas as pl
from jax.experimental.pallas import tpu as pltpu
from jax.experimental.pallas import tpu_sc as plsc

N = 10000
E = 320000
D = 128
NC = 2
NS = 16
K = 80

E_PER_T = E // (NC * NS)

ROW_START = [s * 624 for s in range(16)]
ROW_CNT = [624] * 15 + [N - 15 * 624]


@functools.lru_cache(maxsize=None)
def _mesh():
    return plsc.VectorSubcoreMesh(
        core_axis_name="c", subcore_axis_name="s", num_cores=NC, num_subcores=NS)


@functools.lru_cache(maxsize=None)
def _deg_kernel():
    @functools.partial(
        pl.kernel,
        mesh=_mesh(),
        out_type=(
            jax.ShapeDtypeStruct((N, D), jnp.float32),
            jax.ShapeDtypeStruct((N, D), jnp.float32),
            jax.ShapeDtypeStruct((N, D), jnp.float32),
            jax.ShapeDtypeStruct((N, D), jnp.float32),
        ),
        scratch_types=[
            pltpu.VMEM((K,), jnp.int32),
            pltpu.VMEM((K, D), jnp.float32),
            pltpu.VMEM_SHARED((N, D), jnp.float32),
        ],
    )
    def deg(src_hbm, dst_hbm, zeros_hbm, do_a, do_b, di_a, di_b,
            idx_v, ones_v, acc_sh):
        c = lax.axis_index("c")
        s = lax.axis_index("s")

        one16 = jnp.ones((16,), jnp.float32)

        def fill(i, _):
            for j in range(D // 16):
                ones_v[i, pl.ds(j * 16, 16)] = one16
            return 0

        lax.fori_loop(0, K, fill, 0)

        tile = c * NS + s
        base = tile * E_PER_T

        def phase(idx_hbm, out_a, out_b):
            for sid in range(NS):
                @pl.when(s == sid)
                def _():
                    r0, rn = ROW_START[sid], ROW_CNT[sid]
                    pltpu.sync_copy(zeros_hbm.at[pl.ds(r0, rn)],
                                    acc_sh.at[pl.ds(r0, rn)])

            plsc.subcore_barrier()

            def body(i, _):
                off = base + i * K
                pltpu.sync_copy(idx_hbm.at[pl.ds(off, K)], idx_v)
                pltpu.sync_copy(ones_v, acc_sh.at[idx_v], add=True)
                return 0

            lax.fori_loop(0, E_PER_T // K, body, 0)

            plsc.subcore_barrier()

            for sid in range(NS):
                @pl.when(s == sid)
                def _():
                    r0, rn = ROW_START[sid], ROW_CNT[sid]

                    @pl.when(c == 0)
                    def _():
                        pltpu.sync_copy(acc_sh.at[pl.ds(r0, rn)],
                                        out_a.at[pl.ds(r0, rn)])

                    @pl.when(c == 1)
                    def _():
                        pltpu.sync_copy(acc_sh.at[pl.ds(r0, rn)],
                                        out_b.at[pl.ds(r0, rn)])

        phase(src_hbm, do_a, do_b)
        phase(dst_hbm, di_a, di_b)

    return deg


@functools.lru_cache(maxsize=None)
def _spmm_kernel():
    NCH = E_PER_T // K
    NPAIR = NCH // 2

    @functools.partial(
        pl.kernel,
        mesh=_mesh(),
        out_type=(
            jax.ShapeDtypeStruct((N, D), jnp.float32),
            jax.ShapeDtypeStruct((N, D), jnp.float32),
        ),
        scratch_types=[
            pltpu.VMEM((E_PER_T,), jnp.int32),
            pltpu.VMEM((E_PER_T,), jnp.int32),
            pltpu.VMEM((K,), jnp.int32),
            pltpu.VMEM((K,), jnp.int32),
            pltpu.VMEM((K, D), jnp.float32),
            pltpu.VMEM((K, D), jnp.float32),
            pltpu.VMEM_SHARED((N, D), jnp.float32),
            pltpu.SemaphoreType.DMA,
            pltpu.SemaphoreType.DMA,
        ],
    )
    def spmm(src_hbm, dst_hbm, g_hbm, zeros_hbm, s_a, s_b,
             big_src, big_dst, dsm0, dsm1, rows0, rows1, s_sh, sem0, sem1):
        c = lax.axis_index("c")
        s = lax.axis_index("s")

        for sid in range(NS):
            @pl.when(s == sid)
            def _():
                r0, rn = ROW_START[sid], ROW_CNT[sid]
                pltpu.sync_copy(zeros_hbm.at[pl.ds(r0, rn)], s_sh.at[pl.ds(r0, rn)])

        plsc.subcore_barrier()

        tile = c * NS + s
        base = tile * E_PER_T

        pltpu.sync_copy(src_hbm.at[pl.ds(base, E_PER_T)], big_src)
        pltpu.sync_copy(dst_hbm.at[pl.ds(base, E_PER_T)], big_dst)

        dsm = (dsm0, dsm1)
        rows = (rows0, rows1)
        sems = (sem0, sem1)

        def prep(buf, chunk):
            for j in range(K // 16):
                buf[pl.ds(j * 16, 16)] = big_dst[pl.ds(chunk * K + j * 16, 16)]

        def edge_loop(table):
            prep(dsm[0], 0)
            prep(dsm[1], 1)
            pltpu.async_copy(table.at[big_src.at[pl.ds(0, K)]], rows[0], sems[0])

            def pair(o, _):
                a = 2 * o
                pltpu.make_async_copy(table.at[big_src.at[pl.ds(0, K)]],
                                      rows[0], sems[0]).wait()
                pltpu.async_copy(table.at[big_src.at[pl.ds((a + 1) * K, K)]],
                                 rows[1], sems[1])
                pltpu.sync_copy(rows[0], s_sh.at[dsm[0]], add=True)
                prep(dsm[0], a + 2)
                pltpu.make_async_copy(table.at[big_src.at[pl.ds(0, K)]],
                                      rows[1], sems[1]).wait()
                pltpu.async_copy(table.at[big_src.at[pl.ds((a + 2) * K, K)]],
                                 rows[0], sems[0])
                pltpu.sync_copy(rows[1], s_sh.at[dsm[1]], add=True)

                @pl.when(o < NPAIR - 1)
                def _():
                    prep(dsm[1], a + 3)

                return 0

            lax.fori_loop(0, NPAIR, pair, 0)

            pltpu.make_async_copy(table.at[big_src.at[pl.ds(0, K)]],
                                  rows[0], sems[0]).wait()
            pltpu.sync_copy(rows[0], s_sh.at[dsm[0]], add=True)

        edge_loop(g_hbm)

        plsc.subcore_barrier()

        for sid in range(NS):
            @pl.when(s == sid)
            def _():
                r0, rn = ROW_START[sid], ROW_CNT[sid]

                @pl.when(c == 0)
                def _():
                    pltpu.sync_copy(s_sh.at[pl.ds(r0, rn)], s_a.at[pl.ds(r0, rn)])

                @pl.when(c == 1)
                def _():
                    pltpu.sync_copy(s_sh.at[pl.ds(r0, rn)], s_b.at[pl.ds(r0, rn)])

    return spmm


BLK = 1000
GRID = N // BLK


def _row_spec(w):
    return pl.BlockSpec((BLK, w), lambda i: (i, 0))


def _full_spec(shape):
    return pl.BlockSpec(shape, lambda i: tuple(0 for _ in shape))


def _pre_body(x, wpre, bpre, wc1, do_a, do_b, h_out, g_out):
    h = jnp.dot(x[...], wpre[...], preferred_element_type=jnp.float32) + bpre[...]
    h_out[...] = h
    ra = lax.rsqrt(jnp.maximum(do_a[...] + do_b[...], 1.0))[:, 0:1]
    g_out[...] = ra * jnp.dot(h, wc1[...], preferred_element_type=jnp.float32)


def _mid_body(h_in, s_a, s_b, bc, wnext, do_a, do_b, di_a, di_b, h_out, g_out):
    rb = lax.rsqrt(jnp.maximum(di_a[...] + di_b[...], 1.0))[:, 0:1]
    h = h_in[...] + jnp.maximum(rb * (s_a[...] + s_b[...]) + bc[...], 0.0)
    h_out[...] = h
    ra = lax.rsqrt(jnp.maximum(do_a[...] + do_b[...], 1.0))[:, 0:1]
    g_out[...] = ra * jnp.dot(h, wnext[...], preferred_element_type=jnp.float32)


def _post_body(h_in, s_a, s_b, bc, di_a, di_b, wp1, bp1, wp2, bp2, out):
    rb = lax.rsqrt(jnp.maximum(di_a[...] + di_b[...], 1.0))[:, 0:1]
    h = h_in[...] + jnp.maximum(rb * (s_a[...] + s_b[...]) + bc[...], 0.0)
    h2 = jnp.maximum(
        jnp.dot(h, wp1[...], preferred_element_type=jnp.float32) + bp1[...], 0.0)
    out[...] = jnp.dot(h2, wp2[...], preferred_element_type=jnp.float32) + bp2[...]


def _pre_call(x, wpre, bpre, wc1, do_a, do_b):
    return pl.pallas_call(
        _pre_body,
        grid=(GRID,),
        in_specs=[
            _row_spec(D), _full_spec((D, D)), _full_spec((D,)), _full_spec((D, D)),
            _row_spec(D), _row_spec(D),
        ],
        out_specs=[_row_spec(D), _row_spec(D)],
        out_shape=[
            jax.ShapeDtypeStruct((N, D), jnp.float32),
            jax.ShapeDtypeStruct((N, D), jnp.float32),
        ],
    )(x, wpre, bpre, wc1, do_a, do_b)


def _mid_call(h, s_a, s_b, bc, wnext, do_a, do_b, di_a, di_b):
    return pl.pallas_call(
        _mid_body,
        grid=(GRID,),
        in_specs=[
            _row_spec(D), _row_spec(D), _row_spec(D),
            _full_spec((D,)), _full_spec((D, D)),
            _row_spec(D), _row_spec(D), _row_spec(D), _row_spec(D),
        ],
        out_specs=[_row_spec(D), _row_spec(D)],
        out_shape=[
            jax.ShapeDtypeStruct((N, D), jnp.float32),
            jax.ShapeDtypeStruct((N, D), jnp.float32),
        ],
    )(h, s_a, s_b, bc, wnext, do_a, do_b, di_a, di_b)


def _post_call(h, s_a, s_b, bc, di_a, di_b, wp1, bp1, wp2, bp2):
    return pl.pallas_call(
        _post_body,
        grid=(GRID,),
        in_specs=[
            _row_spec(D), _row_spec(D), _row_spec(D),
            _full_spec((D,)),
            _row_spec(D), _row_spec(D),
            _full_spec((D, D)), _full_spec((D,)),
            _full_spec((D, D)), _full_spec((D,)),
        ],
        out_specs=_row_spec(D),
        out_shape=jax.ShapeDtypeStruct((N, D), jnp.float32),
    )(h, s_a, s_b, bc, di_a, di_b, wp1, bp1, wp2, bp2)


def kernel(x, adj, W_pre, b_pre, Wc1, bc1, Wc2, bc2, Wc3, bc3, Wp1, bp1, Wp2, bp2):
    src = adj[0]
    dst = adj[1]
    zeros128 = jnp.zeros((N, D), jnp.float32)

    do_a, do_b, di_a, di_b = _deg_kernel()(src, dst, zeros128)

    h, g = _pre_call(x, W_pre, b_pre, Wc1, do_a, do_b)

    s_a, s_b = _spmm_kernel()(src, dst, g, zeros128)
    h, g = _mid_call(h, s_a, s_b, bc1, Wc2, do_a, do_b, di_a, di_b)

    s_a, s_b = _spmm_kernel()(src, dst, g, zeros128)
    h, g = _mid_call(h, s_a, s_b, bc2, Wc3, do_a, do_b, di_a, di_b)

    s_a, s_b = _spmm_kernel()(src, dst, g, zeros128)
    out = _post_call(h, s_a, s_b, bc3, di_a, di_b, Wp1, bp1, Wp2, bp2)
    return out


# speedup vs baseline: 13.5916x; 1.1499x over previous
import functools

import jax
import jax.numpy as jnp
from jax import lax
from jax.experimental import pallas as pl
from jax.experimental.pallas import tpu as pltpu
from jax.experimental.pallas import tpu_sc as plsc

N = 10000
E = 320000
D = 128
NC = 2
NS = 16
K = 80

E_PER_T = E // (NC * NS)
NCH = E_PER_T // K
NPAIR = NCH // 2

ROW_START = [s * 624 for s in range(16)]
ROW_CNT = [624] * 15 + [N - 15 * 624]


@functools.lru_cache(maxsize=None)
def _mesh():
    return plsc.VectorSubcoreMesh(
        core_axis_name="c", subcore_axis_name="s", num_cores=NC, num_subcores=NS)


@functools.lru_cache(maxsize=None)
def _deg_kernel():
    @functools.partial(
        pl.kernel,
        mesh=_mesh(),
        out_type=(
            jax.ShapeDtypeStruct((N, D), jnp.float32),
            jax.ShapeDtypeStruct((N, D), jnp.float32),
            jax.ShapeDtypeStruct((N, D), jnp.float32),
            jax.ShapeDtypeStruct((N, D), jnp.float32),
        ),
        scratch_types=[
            pltpu.VMEM((E_PER_T,), jnp.int32),
            pltpu.VMEM((E_PER_T,), jnp.int32),
            pltpu.VMEM((K,), jnp.int32),
            pltpu.VMEM((K,), jnp.int32),
            pltpu.VMEM((K, D), jnp.float32),
            pltpu.VMEM_SHARED((N, D), jnp.float32),
            pltpu.SemaphoreType.DMA,
            pltpu.SemaphoreType.DMA,
        ],
    )
    def deg(src_hbm, dst_hbm, zeros_hbm, do_a, do_b, di_a, di_b,
            big_src, big_dst, ib0, ib1, ones_v, acc_sh, sem0, sem1):
        c = lax.axis_index("c")
        s = lax.axis_index("s")

        one16 = jnp.ones((16,), jnp.float32)

        def fill(i, _):
            for j in range(D // 16):
                ones_v[i, pl.ds(j * 16, 16)] = one16
            return 0

        lax.fori_loop(0, K, fill, 0)

        tile = c * NS + s
        base = tile * E_PER_T

        pltpu.sync_copy(src_hbm.at[pl.ds(base, E_PER_T)], big_src)
        pltpu.sync_copy(dst_hbm.at[pl.ds(base, E_PER_T)], big_dst)

        def phase(big_idx, out_a, out_b):
            for sid in range(NS):
                @pl.when(s == sid)
                def _():
                    r0, rn = ROW_START[sid], ROW_CNT[sid]
                    pltpu.sync_copy(zeros_hbm.at[pl.ds(r0, rn)],
                                    acc_sh.at[pl.ds(r0, rn)])

            plsc.subcore_barrier()

            def prep(buf, chunk):
                for j in range(K // 16):
                    buf[pl.ds(j * 16, 16)] = big_idx[pl.ds(chunk * K + j * 16, 16)]

            def scat(buf, sem):
                pltpu.async_copy(ones_v, acc_sh.at[buf], sem, add=True)

            def drain(buf, sem):
                pltpu.make_async_copy(ones_v, acc_sh.at[buf], sem).wait()

            prep(ib0, 0)
            scat(ib0, sem0)
            prep(ib1, 1)
            scat(ib1, sem1)

            def pair(o, _):
                a = 2 * o
                drain(ib0, sem0)
                prep(ib0, a + 2)
                scat(ib0, sem0)
                drain(ib1, sem1)

                @pl.when(o < NPAIR - 1)
                def _():
                    prep(ib1, a + 3)
                    scat(ib1, sem1)

                return 0

            lax.fori_loop(0, NPAIR, pair, 0)
            drain(ib0, sem0)

            plsc.subcore_barrier()

            for sid in range(NS):
                @pl.when(s == sid)
                def _():
                    r0, rn = ROW_START[sid], ROW_CNT[sid]

                    @pl.when(c == 0)
                    def _():
                        pltpu.sync_copy(acc_sh.at[pl.ds(r0, rn)],
                                        out_a.at[pl.ds(r0, rn)])

                    @pl.when(c == 1)
                    def _():
                        pltpu.sync_copy(acc_sh.at[pl.ds(r0, rn)],
                                        out_b.at[pl.ds(r0, rn)])

        phase(big_src, do_a, do_b)
        phase(big_dst, di_a, di_b)

    return deg


@functools.lru_cache(maxsize=None)
def _spmm_kernel():
    @functools.partial(
        pl.kernel,
        mesh=_mesh(),
        out_type=(
            jax.ShapeDtypeStruct((N, D), jnp.float32),
            jax.ShapeDtypeStruct((N, D), jnp.float32),
        ),
        scratch_types=[
            pltpu.VMEM((E_PER_T,), jnp.int32),
            pltpu.VMEM((E_PER_T,), jnp.int32),
            pltpu.VMEM((K,), jnp.int32),
            pltpu.VMEM((K,), jnp.int32),
            pltpu.VMEM((K, D), jnp.float32),
            pltpu.VMEM((K, D), jnp.float32),
            pltpu.VMEM_SHARED((N, D), jnp.float32),
            pltpu.SemaphoreType.DMA,
            pltpu.SemaphoreType.DMA,
        ],
    )
    def spmm(src_hbm, dst_hbm, g_hbm, zeros_hbm, s_a, s_b,
             big_src, big_dst, dsm0, dsm1, rows0, rows1, s_sh, sem0, sem1):
        c = lax.axis_index("c")
        s = lax.axis_index("s")

        for sid in range(NS):
            @pl.when(s == sid)
            def _():
                r0, rn = ROW_START[sid], ROW_CNT[sid]
                pltpu.sync_copy(zeros_hbm.at[pl.ds(r0, rn)], s_sh.at[pl.ds(r0, rn)])

        plsc.subcore_barrier()

        tile = c * NS + s
        base = tile * E_PER_T

        pltpu.sync_copy(src_hbm.at[pl.ds(base, E_PER_T)], big_src)
        pltpu.sync_copy(dst_hbm.at[pl.ds(base, E_PER_T)], big_dst)

        dsm = (dsm0, dsm1)
        rows = (rows0, rows1)
        sems = (sem0, sem1)

        def prep(buf, chunk):
            for j in range(K // 16):
                buf[pl.ds(j * 16, 16)] = big_dst[pl.ds(chunk * K + j * 16, 16)]

        def edge_loop(table):
            prep(dsm[0], 0)
            prep(dsm[1], 1)
            pltpu.async_copy(table.at[big_src.at[pl.ds(0, K)]], rows[0], sems[0])

            def pair(o, _):
                a = 2 * o
                pltpu.make_async_copy(table.at[big_src.at[pl.ds(0, K)]],
                                      rows[0], sems[0]).wait()
                pltpu.async_copy(table.at[big_src.at[pl.ds((a + 1) * K, K)]],
                                 rows[1], sems[1])
                pltpu.sync_copy(rows[0], s_sh.at[dsm[0]], add=True)
                prep(dsm[0], a + 2)
                pltpu.make_async_copy(table.at[big_src.at[pl.ds(0, K)]],
                                      rows[1], sems[1]).wait()
                pltpu.async_copy(table.at[big_src.at[pl.ds((a + 2) * K, K)]],
                                 rows[0], sems[0])
                pltpu.sync_copy(rows[1], s_sh.at[dsm[1]], add=True)

                @pl.when(o < NPAIR - 1)
                def _():
                    prep(dsm[1], a + 3)

                return 0

            lax.fori_loop(0, NPAIR, pair, 0)

            pltpu.make_async_copy(table.at[big_src.at[pl.ds(0, K)]],
                                  rows[0], sems[0]).wait()
            pltpu.sync_copy(rows[0], s_sh.at[dsm[0]], add=True)

        edge_loop(g_hbm)

        plsc.subcore_barrier()

        for sid in range(NS):
            @pl.when(s == sid)
            def _():
                r0, rn = ROW_START[sid], ROW_CNT[sid]

                @pl.when(c == 0)
                def _():
                    pltpu.sync_copy(s_sh.at[pl.ds(r0, rn)], s_a.at[pl.ds(r0, rn)])

                @pl.when(c == 1)
                def _():
                    pltpu.sync_copy(s_sh.at[pl.ds(r0, rn)], s_b.at[pl.ds(r0, rn)])

    return spmm


BLK = 1000
GRID = N // BLK


def _row_spec(w):
    return pl.BlockSpec((BLK, w), lambda i: (i, 0))


def _full_spec(shape):
    return pl.BlockSpec(shape, lambda i: tuple(0 for _ in shape))


def _pre_body(x, wpre, bpre, wc1, do_a, do_b, h_out, g_out):
    h = jnp.dot(x[...], wpre[...], preferred_element_type=jnp.float32) + bpre[...]
    h_out[...] = h
    ra = lax.rsqrt(jnp.maximum(do_a[...] + do_b[...], 1.0))[:, 0:1]
    g_out[...] = ra * jnp.dot(h, wc1[...], preferred_element_type=jnp.float32)


def _mid_body(h_in, s_a, s_b, bc, wnext, do_a, do_b, di_a, di_b, h_out, g_out):
    rb = lax.rsqrt(jnp.maximum(di_a[...] + di_b[...], 1.0))[:, 0:1]
    h = h_in[...] + jnp.maximum(rb * (s_a[...] + s_b[...]) + bc[...], 0.0)
    h_out[...] = h
    ra = lax.rsqrt(jnp.maximum(do_a[...] + do_b[...], 1.0))[:, 0:1]
    g_out[...] = ra * jnp.dot(h, wnext[...], preferred_element_type=jnp.float32)


def _post_body(h_in, s_a, s_b, bc, di_a, di_b, wp1, bp1, wp2, bp2, out):
    rb = lax.rsqrt(jnp.maximum(di_a[...] + di_b[...], 1.0))[:, 0:1]
    h = h_in[...] + jnp.maximum(rb * (s_a[...] + s_b[...]) + bc[...], 0.0)
    h2 = jnp.maximum(
        jnp.dot(h, wp1[...], preferred_element_type=jnp.float32) + bp1[...], 0.0)
    out[...] = jnp.dot(h2, wp2[...], preferred_element_type=jnp.float32) + bp2[...]


def _pre_call(x, wpre, bpre, wc1, do_a, do_b):
    return pl.pallas_call(
        _pre_body,
        grid=(GRID,),
        in_specs=[
            _row_spec(D), _full_spec((D, D)), _full_spec((D,)), _full_spec((D, D)),
            _row_spec(D), _row_spec(D),
        ],
        out_specs=[_row_spec(D), _row_spec(D)],
        out_shape=[
            jax.ShapeDtypeStruct((N, D), jnp.float32),
            jax.ShapeDtypeStruct((N, D), jnp.float32),
        ],
    )(x, wpre, bpre, wc1, do_a, do_b)


def _mid_call(h, s_a, s_b, bc, wnext, do_a, do_b, di_a, di_b):
    return pl.pallas_call(
        _mid_body,
        grid=(GRID,),
        in_specs=[
            _row_spec(D), _row_spec(D), _row_spec(D),
            _full_spec((D,)), _full_spec((D, D)),
            _row_spec(D), _row_spec(D), _row_spec(D), _row_spec(D),
        ],
        out_specs=[_row_spec(D), _row_spec(D)],
        out_shape=[
            jax.ShapeDtypeStruct((N, D), jnp.float32),
            jax.ShapeDtypeStruct((N, D), jnp.float32),
        ],
    )(h, s_a, s_b, bc, wnext, do_a, do_b, di_a, di_b)


def _post_call(h, s_a, s_b, bc, di_a, di_b, wp1, bp1, wp2, bp2):
    return pl.pallas_call(
        _post_body,
        grid=(GRID,),
        in_specs=[
            _row_spec(D), _row_spec(D), _row_spec(D),
            _full_spec((D,)),
            _row_spec(D), _row_spec(D),
            _full_spec((D, D)), _full_spec((D,)),
            _full_spec((D, D)), _full_spec((D,)),
        ],
        out_specs=_row_spec(D),
        out_shape=jax.ShapeDtypeStruct((N, D), jnp.float32),
    )(h, s_a, s_b, bc, di_a, di_b, wp1, bp1, wp2, bp2)


def kernel(x, adj, W_pre, b_pre, Wc1, bc1, Wc2, bc2, Wc3, bc3, Wp1, bp1, Wp2, bp2):
    src = adj[0]
    dst = adj[1]
    zeros128 = jnp.zeros((N, D), jnp.float32)

    do_a, do_b, di_a, di_b = _deg_kernel()(src, dst, zeros128)

    h, g = _pre_call(x, W_pre, b_pre, Wc1, do_a, do_b)

    s_a, s_b = _spmm_kernel()(src, dst, g, zeros128)
    h, g = _mid_call(h, s_a, s_b, bc1, Wc2, do_a, do_b, di_a, di_b)

    s_a, s_b = _spmm_kernel()(src, dst, g, zeros128)
    h, g = _mid_call(h, s_a, s_b, bc2, Wc3, do_a, do_b, di_a, di_b)

    s_a, s_b = _spmm_kernel()(src, dst, g, zeros128)
    out = _post_call(h, s_a, s_b, bc3, di_a, di_b, Wp1, bp1, Wp2, bp2)
    return out
